# baseline (speedup 1.0000x reference)
import functools

import jax
import jax.numpy as jnp
from jax import lax
from jax.experimental import pallas as pl
from jax.experimental.pallas import tpu as pltpu
from jax.experimental.pallas import tpu_sc as plsc

N = 10000
NP = 10240
D = 128
E = 320000
NC = 2
NS = 16
NW = NC * NS
CHUNK = 64
CPW = 160
EPAD = NW * CPW * CHUNK
IB = 10
NBLK = CPW // IB
RB = 4
ROWS_PER_TILE = NP // NS


def _sc_agg_body(with_counts, table, src_idx, dst_idx, zeros_tab, zeros_cnt,
                 ones_hbm, agg_out, cnt_out,
                 agg_sh, cnt_sh, src_a, dst_a, src_b, dst_b,
                 rows0, rows1, rows2, rows3, ones_v,
                 g0, g1, g2, g3, s0, s1, s2, s3, c0, c1, c2, c3, idx_sem):
    c = lax.axis_index("c")
    s = lax.axis_index("s")
    w = c * NS + s

    src_blks = (src_a, src_b)
    dst_blks = (dst_a, dst_b)
    rows = (rows0, rows1, rows2, rows3)
    g_sems = (g0, g1, g2, g3)
    s_sems = (s0, s1, s2, s3)
    c_sems = (c0, c1, c2, c3)

    def idx_refill(m, p, start):
        op = pltpu.async_copy if start else (
            lambda s_, d_, m_: pltpu.make_async_copy(s_, d_, m_).wait())
        op(src_idx.at[w, m], src_blks[p], idx_sem)
        op(dst_idx.at[w, m], dst_blks[p], idx_sem)

    def gather(p, jj, b, start):
        ref = table.at[src_blks[p].at[jj]]
        if start:
            pltpu.async_copy(ref, rows[b], g_sems[b])
        else:
            pltpu.make_async_copy(ref, rows[b], g_sems[b]).wait()

    def scatter(p, jj, b, start):
        ref = agg_sh.at[dst_blks[p].at[jj]]
        if start:
            pltpu.async_copy(rows[b], ref, s_sems[b], add=True)
            if with_counts:
                pltpu.async_copy(ones_v, cnt_sh.at[dst_blks[p].at[jj]],
                                 c_sems[b], add=True)
        else:
            pltpu.make_async_copy(rows[b], ref, s_sems[b]).wait()
            if with_counts:
                pltpu.make_async_copy(ones_v, cnt_sh.at[dst_blks[p].at[jj]],
                                      c_sems[b]).wait()

    r0 = s * ROWS_PER_TILE
    pltpu.sync_copy(zeros_tab.at[pl.ds(r0, ROWS_PER_TILE)],
                    agg_sh.at[pl.ds(r0, ROWS_PER_TILE)])
    if with_counts:
        pltpu.sync_copy(ones_hbm, ones_v)
        pltpu.sync_copy(zeros_cnt.at[pl.ds(r0, ROWS_PER_TILE)],
                        cnt_sh.at[pl.ds(r0, ROWS_PER_TILE)])

    idx_refill(0, 0, True)
    idx_refill(0, 0, False)
    idx_refill(1, 1, True)
    plsc.subcore_barrier()
    gather(0, 0, 0, True)
    gather(0, 1, 1, True)

    def step(kk, carry):
        for p in range(2):
            m = 2 * kk + p
            for jj in range(IB):
                j = m * IB + jj
                b = (2 * p + jj) % RB
                bg = (2 * p + jj + 2) % RB
                gather(p, jj, b, False)
                scatter(p, jj, b, True)
                if jj >= 2:
                    scatter(p, jj - 2, bg, False)
                else:
                    @pl.when(m >= 1)
                    def _():
                        scatter(p, jj, bg, False)
                if jj == IB - 2:
                    @pl.when(m + 1 < NBLK)
                    def _():
                        idx_refill(m + 1, 1 - p, False)
                if jj + 2 < IB:
                    @pl.when(j + 2 < CPW)
                    def _():
                        gather(p, jj + 2, bg, True)
                else:
                    @pl.when(j + 2 < CPW)
                    def _():
                        gather(1 - p, jj + 2 - IB, bg, True)
            @pl.when(m + 2 < NBLK)
            def _():
                idx_refill(m + 2, p, True)
        return carry

    lax.fori_loop(0, NBLK // 2, step, 0)

    scatter(1, IB - 2, (2 + IB - 2) % RB, False)
    scatter(1, IB - 1, (2 + IB - 1) % RB, False)

    plsc.subcore_barrier()

    pltpu.sync_copy(agg_sh.at[pl.ds(r0, ROWS_PER_TILE)],
                    agg_out.at[c, pl.ds(r0, ROWS_PER_TILE)])
    if with_counts:
        pltpu.sync_copy(cnt_sh.at[pl.ds(r0, ROWS_PER_TILE)],
                        cnt_out.at[pl.ds(c * NP + r0, ROWS_PER_TILE)])


def _make_sc_agg(with_counts):
    return pl.kernel(
        functools.partial(_sc_agg_body, with_counts),
        out_type=(jax.ShapeDtypeStruct((NC, NP, D), jnp.float32),
                  jax.ShapeDtypeStruct((NC * NP,), jnp.float32)),
        mesh=plsc.VectorSubcoreMesh(core_axis_name="c", subcore_axis_name="s"),
        scratch_types=[
            pltpu.VMEM_SHARED((NP, D), jnp.float32),
            pltpu.VMEM_SHARED((NP,), jnp.float32),
            pltpu.VMEM((IB, CHUNK), jnp.int32),
            pltpu.VMEM((IB, CHUNK), jnp.int32),
            pltpu.VMEM((IB, CHUNK), jnp.int32),
            pltpu.VMEM((IB, CHUNK), jnp.int32),
            pltpu.VMEM((CHUNK, D), jnp.float32),
            pltpu.VMEM((CHUNK, D), jnp.float32),
            pltpu.VMEM((CHUNK, D), jnp.float32),
            pltpu.VMEM((CHUNK, D), jnp.float32),
            pltpu.VMEM((CHUNK,), jnp.float32),
            pltpu.SemaphoreType.DMA, pltpu.SemaphoreType.DMA,
            pltpu.SemaphoreType.DMA, pltpu.SemaphoreType.DMA,
            pltpu.SemaphoreType.DMA, pltpu.SemaphoreType.DMA,
            pltpu.SemaphoreType.DMA, pltpu.SemaphoreType.DMA,
            pltpu.SemaphoreType.DMA, pltpu.SemaphoreType.DMA,
            pltpu.SemaphoreType.DMA, pltpu.SemaphoreType.DMA,
            pltpu.SemaphoreType.DMA,
        ],
    )


_sc_agg_cnt = _make_sc_agg(True)
_sc_agg_plain = _make_sc_agg(False)


_NT = (((1,), (1,)), ((), ()))


def _tc_pre_body(x_ref, wl_ref, wr_ref, b_ref, xl_ref, xrb_ref):
    x = x_ref[...]
    xl_ref[...] = lax.dot_general(x, wl_ref[...], _NT,
                                  preferred_element_type=jnp.float32)
    xrb_ref[...] = lax.dot_general(x, wr_ref[...], _NT,
                                   preferred_element_type=jnp.float32) + b_ref[...]


_tc_pre = pl.pallas_call(
    _tc_pre_body,
    out_shape=(jax.ShapeDtypeStruct((NP, D), jnp.float32),
               jax.ShapeDtypeStruct((NP, D), jnp.float32)),
)


def _tc_mid_body(agg_ref, rc_ref, xrb_ref, wl_ref, wr_ref, b_ref,
                 xl_ref, xrb2_ref):
    t = (agg_ref[0] + agg_ref[1]) * rc_ref[...] + xrb_ref[...]
    h = jnp.maximum(t, 0.0)
    xl_ref[...] = lax.dot_general(h, wl_ref[...], _NT,
                                  preferred_element_type=jnp.float32)
    xrb2_ref[...] = lax.dot_general(h, wr_ref[...], _NT,
                                    preferred_element_type=jnp.float32) + b_ref[...]


_tc_mid = pl.pallas_call(
    _tc_mid_body,
    out_shape=(jax.ShapeDtypeStruct((NP, D), jnp.float32),
               jax.ShapeDtypeStruct((NP, D), jnp.float32)),
)


def _tc_out_body(agg_ref, rc_ref, xrb_ref, out_ref):
    out_ref[...] = (agg_ref[0] + agg_ref[1]) * rc_ref[...] + xrb_ref[...]


_tc_out = pl.pallas_call(
    _tc_out_body,
    out_shape=jax.ShapeDtypeStruct((NP, D), jnp.float32),
)


def kernel(x, edge_index, W1l, b1, W1r, W2l, b2, W2r):
    npad = EPAD - E
    pad_src = (jnp.arange(npad, dtype=jnp.int32) * 7) % N
    pad_dst = N + (jnp.arange(npad, dtype=jnp.int32) % (NP - N))
    src = jnp.concatenate([edge_index[0].astype(jnp.int32), pad_src])
    dst = jnp.concatenate([edge_index[1].astype(jnp.int32), pad_dst])
    src = src.reshape(NW, NBLK, IB, CHUNK)
    dst = dst.reshape(NW, NBLK, IB, CHUNK)
    zeros_tab = jnp.zeros((NP, D), jnp.float32)
    zeros_cnt = jnp.zeros((NP,), jnp.float32)
    ones = jnp.ones((CHUNK,), jnp.float32)

    xp = jnp.pad(x, ((0, NP - N), (0, 0)))
    xl1, xr1b = _tc_pre(xp, W1l, W1r, b1.reshape(1, D))
    agg1, cnt = _sc_agg_cnt(xl1, src, dst, zeros_tab, zeros_cnt, ones)
    rc = (1.0 / jnp.maximum(cnt[:NP] + cnt[NP:], 1.0)).reshape(NP, 1)
    xl2, xr2b = _tc_mid(agg1, rc, xr1b, W2l, W2r, b2.reshape(1, D))
    agg2, _ = _sc_agg_plain(xl2, src, dst, zeros_tab, zeros_cnt, ones)
    return _tc_out(agg2, rc, xr2b)[:N]


# speedup vs baseline: 1.1491x; 1.1491x over previous
import jax
import jax.numpy as jnp
from jax import lax
from jax.experimental import pallas as pl
from jax.experimental.pallas import tpu as pltpu
from jax.experimental.pallas import tpu_sc as plsc

N = 10000
NP = 10240
D = 128
E = 320000
NC = 2
NS = 16
NW = NC * NS
CHUNK = 125
CPW = E // (NW * CHUNK)
IB = 10
NBLK = CPW // IB
ROWS_PER_TILE = NP // NS


def _sc_agg_body(table, src_idx, dst_idx, zeros_tab,
                 agg_out,
                 agg_sh, src_a, dst_a, src_b, dst_b, rows0, rows1,
                 sem0, sem1, idx_sem):
    c = lax.axis_index("c")
    s = lax.axis_index("s")
    w = c * NS + s

    src_blks = (src_a, src_b)
    dst_blks = (dst_a, dst_b)
    rows = (rows0, rows1)
    sems = (sem0, sem1)

    def idx_refill(m, p, start):
        op = pltpu.async_copy if start else (
            lambda s_, d_, m_: pltpu.make_async_copy(s_, d_, m_).wait())
        op(src_idx.at[w, m], src_blks[p], idx_sem)
        op(dst_idx.at[w, m], dst_blks[p], idx_sem)

    def gather(p, jj, b, start):
        src_ref = table.at[src_blks[p].at[jj]]
        if start:
            pltpu.async_copy(src_ref, rows[b], sems[b])
        else:
            pltpu.make_async_copy(src_ref, rows[b], sems[b]).wait()

    r0 = s * ROWS_PER_TILE
    pltpu.sync_copy(zeros_tab.at[pl.ds(r0, ROWS_PER_TILE)],
                    agg_sh.at[pl.ds(r0, ROWS_PER_TILE)])

    idx_refill(0, 0, True)
    idx_refill(0, 0, False)
    idx_refill(1, 1, True)
    plsc.subcore_barrier()
    gather(0, 0, 0, True)
    gather(0, 1, 1, True)

    def step(kk, carry):
        for p in range(2):
            m = 2 * kk + p
            for jj in range(IB):
                j = m * IB + jj
                b = jj % 2
                gather(p, jj, b, False)
                pltpu.sync_copy(rows[b], agg_sh.at[dst_blks[p].at[jj]],
                                add=True)
                if jj == IB - 2:
                    @pl.when(m + 1 < NBLK)
                    def _():
                        idx_refill(m + 1, 1 - p, False)
                if jj + 2 < IB:
                    @pl.when(j + 2 < CPW)
                    def _():
                        gather(p, jj + 2, b, True)
                else:
                    @pl.when(j + 2 < CPW)
                    def _():
                        gather(1 - p, jj + 2 - IB, b, True)
            @pl.when(m + 2 < NBLK)
            def _():
                idx_refill(m + 2, p, True)
        return carry

    lax.fori_loop(0, NBLK // 2, step, 0)

    plsc.subcore_barrier()

    pltpu.sync_copy(agg_sh.at[pl.ds(r0, ROWS_PER_TILE)],
                    agg_out.at[c, pl.ds(r0, ROWS_PER_TILE)])


_sc_agg = pl.kernel(
    _sc_agg_body,
    out_type=jax.ShapeDtypeStruct((NC, NP, D), jnp.float32),
    mesh=plsc.VectorSubcoreMesh(core_axis_name="c", subcore_axis_name="s"),
    scratch_types=[
        pltpu.VMEM_SHARED((NP, D), jnp.float32),
        pltpu.VMEM((IB, CHUNK), jnp.int32),
        pltpu.VMEM((IB, CHUNK), jnp.int32),
        pltpu.VMEM((IB, CHUNK), jnp.int32),
        pltpu.VMEM((IB, CHUNK), jnp.int32),
        pltpu.VMEM((CHUNK, D), jnp.float32),
        pltpu.VMEM((CHUNK, D), jnp.float32),
        pltpu.SemaphoreType.DMA,
        pltpu.SemaphoreType.DMA,
        pltpu.SemaphoreType.DMA,
    ],
)


def _sc_cnt_body(dst_idx, zeros_cnt, ones_hbm,
                 cnt_out,
                 cnt_sh, dst_v, ones_v):
    c = lax.axis_index("c")
    s = lax.axis_index("s")
    w = c * NS + s

    pltpu.sync_copy(dst_idx.at[w], dst_v)
    pltpu.sync_copy(ones_hbm, ones_v)
    r0 = s * ROWS_PER_TILE
    pltpu.sync_copy(zeros_cnt.at[pl.ds(r0, ROWS_PER_TILE)],
                    cnt_sh.at[pl.ds(r0, ROWS_PER_TILE)])
    plsc.subcore_barrier()

    def step(m, carry):
        for jj in range(IB):
            pltpu.sync_copy(ones_v, cnt_sh.at[dst_v.at[m, jj]], add=True)
        return carry

    lax.fori_loop(0, NBLK, step, 0)

    plsc.subcore_barrier()
    pltpu.sync_copy(cnt_sh.at[pl.ds(r0, ROWS_PER_TILE)],
                    cnt_out.at[pl.ds(c * NP + r0, ROWS_PER_TILE)])


_sc_cnt = pl.kernel(
    _sc_cnt_body,
    out_type=jax.ShapeDtypeStruct((NC * NP,), jnp.float32),
    mesh=plsc.VectorSubcoreMesh(core_axis_name="c", subcore_axis_name="s"),
    scratch_types=[
        pltpu.VMEM_SHARED((NP,), jnp.float32),
        pltpu.VMEM((NBLK, IB, CHUNK), jnp.int32),
        pltpu.VMEM((CHUNK,), jnp.float32),
    ],
)


_NT = (((1,), (1,)), ((), ()))


def _tc_pre_body(x_ref, wl_ref, wr_ref, b_ref, xl_ref, xrb_ref):
    x = x_ref[...]
    xl_ref[:N] = lax.dot_general(x, wl_ref[...], _NT,
                                 preferred_element_type=jnp.float32)
    xrb_ref[:N] = lax.dot_general(x, wr_ref[...], _NT,
                                  preferred_element_type=jnp.float32) + b_ref[...]


_tc_pre = pl.pallas_call(
    _tc_pre_body,
    out_shape=(jax.ShapeDtypeStruct((NP, D), jnp.float32),
               jax.ShapeDtypeStruct((NP, D), jnp.float32)),
)


def _tc_mid_body(agg_ref, rc_ref, xrb_ref, wl_ref, wr_ref, b_ref,
                 xl_ref, xrb2_ref):
    t = (agg_ref[0, :N] + agg_ref[1, :N]) * rc_ref[...] + xrb_ref[:N]
    h = jnp.maximum(t, 0.0)
    xl_ref[:N] = lax.dot_general(h, wl_ref[...], _NT,
                                 preferred_element_type=jnp.float32)
    xrb2_ref[:N] = lax.dot_general(h, wr_ref[...], _NT,
                                   preferred_element_type=jnp.float32) + b_ref[...]


_tc_mid = pl.pallas_call(
    _tc_mid_body,
    out_shape=(jax.ShapeDtypeStruct((NP, D), jnp.float32),
               jax.ShapeDtypeStruct((NP, D), jnp.float32)),
)


def _tc_out_body(agg_ref, rc_ref, xrb_ref, out_ref):
    out_ref[...] = (agg_ref[0, :N] + agg_ref[1, :N]) * rc_ref[...] + xrb_ref[:N]


_tc_out = pl.pallas_call(
    _tc_out_body,
    out_shape=jax.ShapeDtypeStruct((N, D), jnp.float32),
)


def kernel(x, edge_index, W1l, b1, W1r, W2l, b2, W2r):
    src = edge_index[0].astype(jnp.int32).reshape(NW, NBLK, IB, CHUNK)
    dst = edge_index[1].astype(jnp.int32).reshape(NW, NBLK, IB, CHUNK)
    zeros_tab = jnp.zeros((NP, D), jnp.float32)
    zeros_cnt = jnp.zeros((NP,), jnp.float32)
    ones = jnp.ones((CHUNK,), jnp.float32)

    cnt = _sc_cnt(dst, zeros_cnt, ones)
    rc = (1.0 / jnp.maximum(cnt[:N] + cnt[NP:NP + N], 1.0)).reshape(N, 1)
    xl1, xr1b = _tc_pre(x, W1l, W1r, b1.reshape(1, D))
    agg1 = _sc_agg(xl1, src, dst, zeros_tab)
    xl2, xr2b = _tc_mid(agg1, rc, xr1b, W2l, W2r, b2.reshape(1, D))
    agg2 = _sc_agg(xl2, src, dst, zeros_tab)
    return _tc_out(agg2, rc, xr2b)


# speedup vs baseline: 1.1651x; 1.0139x over previous
import jax
import jax.numpy as jnp
from jax import lax
from jax.experimental import pallas as pl
from jax.experimental.pallas import tpu as pltpu
from jax.experimental.pallas import tpu_sc as plsc

N = 10000
NP = 10240
D = 128
E = 320000
NC = 2
NS = 16
NW = NC * NS
CHUNK = 125
CPW = E // (NW * CHUNK)
IB = 10
NBLK = CPW // IB
ROWS_PER_TILE = NP // NS


def _sc_agg_body(table, src_idx, dst_idx, zeros_tab,
                 agg_out,
                 agg_sh, src_a, dst_a, src_b, dst_b, rows0, rows1,
                 sem0, sem1, idx_sem):
    c = lax.axis_index("c")
    s = lax.axis_index("s")
    w = c * NS + s

    src_blks = (src_a, src_b)
    dst_blks = (dst_a, dst_b)
    rows = (rows0, rows1)
    sems = (sem0, sem1)

    def idx_refill(m, p, start):
        op = pltpu.async_copy if start else (
            lambda s_, d_, m_: pltpu.make_async_copy(s_, d_, m_).wait())
        op(src_idx.at[w, m], src_blks[p], idx_sem)
        op(dst_idx.at[w, m], dst_blks[p], idx_sem)

    def gather(p, jj, b, start):
        src_ref = table.at[src_blks[p].at[jj]]
        if start:
            pltpu.async_copy(src_ref, rows[b], sems[b])
        else:
            pltpu.make_async_copy(src_ref, rows[b], sems[b]).wait()

    r0 = s * ROWS_PER_TILE
    pltpu.sync_copy(zeros_tab.at[pl.ds(r0, ROWS_PER_TILE)],
                    agg_sh.at[pl.ds(r0, ROWS_PER_TILE)])

    idx_refill(0, 0, True)
    idx_refill(0, 0, False)
    idx_refill(1, 1, True)
    plsc.subcore_barrier()
    gather(0, 0, 0, True)
    gather(0, 1, 1, True)

    def step(kk, carry):
        for p in range(2):
            m = 2 * kk + p
            for jj in range(IB):
                j = m * IB + jj
                b = jj % 2
                gather(p, jj, b, False)
                pltpu.sync_copy(rows[b], agg_sh.at[dst_blks[p].at[jj]],
                                add=True)
                if jj == IB - 2:
                    @pl.when(m + 1 < NBLK)
                    def _():
                        idx_refill(m + 1, 1 - p, False)
                if jj + 2 < IB:
                    @pl.when(j + 2 < CPW)
                    def _():
                        gather(p, jj + 2, b, True)
                else:
                    @pl.when(j + 2 < CPW)
                    def _():
                        gather(1 - p, jj + 2 - IB, b, True)
            @pl.when(m + 2 < NBLK)
            def _():
                idx_refill(m + 2, p, True)
        return carry

    lax.fori_loop(0, NBLK // 2, step, 0)

    plsc.subcore_barrier()

    pltpu.sync_copy(agg_sh.at[pl.ds(r0, ROWS_PER_TILE)],
                    agg_out.at[c, pl.ds(r0, ROWS_PER_TILE)])


_sc_agg = pl.kernel(
    _sc_agg_body,
    out_type=jax.ShapeDtypeStruct((NC, NP, D), jnp.float32),
    mesh=plsc.VectorSubcoreMesh(core_axis_name="c", subcore_axis_name="s"),
    scratch_types=[
        pltpu.VMEM_SHARED((NP, D), jnp.float32),
        pltpu.VMEM((IB, CHUNK), jnp.int32),
        pltpu.VMEM((IB, CHUNK), jnp.int32),
        pltpu.VMEM((IB, CHUNK), jnp.int32),
        pltpu.VMEM((IB, CHUNK), jnp.int32),
        pltpu.VMEM((CHUNK, D), jnp.float32),
        pltpu.VMEM((CHUNK, D), jnp.float32),
        pltpu.SemaphoreType.DMA,
        pltpu.SemaphoreType.DMA,
        pltpu.SemaphoreType.DMA,
    ],
)


def _sc_cnt_body(dst_idx, zeros_cnt, ones_hbm,
                 cnt_out,
                 cnt_sh, dst_v, ones_v, sem):
    c = lax.axis_index("c")
    s = lax.axis_index("s")
    w = c * NS + s

    pltpu.sync_copy(dst_idx.at[w], dst_v)
    pltpu.sync_copy(ones_hbm, ones_v)
    r0 = s * ROWS_PER_TILE
    pltpu.sync_copy(zeros_cnt.at[pl.ds(r0, ROWS_PER_TILE)],
                    cnt_sh.at[pl.ds(r0, ROWS_PER_TILE)])
    plsc.subcore_barrier()

    def step(m, carry):
        for jj in range(IB):
            pltpu.async_copy(ones_v, cnt_sh.at[dst_v.at[m, jj]], sem, add=True)
        for jj in range(IB):
            pltpu.make_async_copy(ones_v, cnt_sh.at[dst_v.at[m, jj]], sem).wait()
        return carry

    lax.fori_loop(0, NBLK, step, 0)

    plsc.subcore_barrier()
    pltpu.sync_copy(cnt_sh.at[pl.ds(r0, ROWS_PER_TILE)],
                    cnt_out.at[pl.ds(c * NP + r0, ROWS_PER_TILE)])


_sc_cnt = pl.kernel(
    _sc_cnt_body,
    out_type=jax.ShapeDtypeStruct((NC * NP,), jnp.float32),
    mesh=plsc.VectorSubcoreMesh(core_axis_name="c", subcore_axis_name="s"),
    scratch_types=[
        pltpu.VMEM_SHARED((NP,), jnp.float32),
        pltpu.VMEM((NBLK, IB, CHUNK), jnp.int32),
        pltpu.VMEM((CHUNK,), jnp.float32),
        pltpu.SemaphoreType.DMA,
    ],
)


_NT = (((1,), (1,)), ((), ()))


def _tc_pre_body(x_ref, wl_ref, wr_ref, b_ref, xl_ref, xrb_ref):
    x = x_ref[...]
    xl_ref[:N] = lax.dot_general(x, wl_ref[...], _NT,
                                 preferred_element_type=jnp.float32)
    xrb_ref[:N] = lax.dot_general(x, wr_ref[...], _NT,
                                  preferred_element_type=jnp.float32) + b_ref[...]


_tc_pre = pl.pallas_call(
    _tc_pre_body,
    out_shape=(jax.ShapeDtypeStruct((NP, D), jnp.float32),
               jax.ShapeDtypeStruct((NP, D), jnp.float32)),
)


def _tc_mid_body(agg_ref, rc_ref, xrb_ref, wl_ref, wr_ref, b_ref,
                 xl_ref, xrb2_ref):
    t = (agg_ref[0, :N] + agg_ref[1, :N]) * rc_ref[...] + xrb_ref[:N]
    h = jnp.maximum(t, 0.0)
    xl_ref[:N] = lax.dot_general(h, wl_ref[...], _NT,
                                 preferred_element_type=jnp.float32)
    xrb2_ref[:N] = lax.dot_general(h, wr_ref[...], _NT,
                                   preferred_element_type=jnp.float32) + b_ref[...]


_tc_mid = pl.pallas_call(
    _tc_mid_body,
    out_shape=(jax.ShapeDtypeStruct((NP, D), jnp.float32),
               jax.ShapeDtypeStruct((NP, D), jnp.float32)),
)


def _tc_out_body(agg_ref, rc_ref, xrb_ref, out_ref):
    out_ref[...] = (agg_ref[0, :N] + agg_ref[1, :N]) * rc_ref[...] + xrb_ref[:N]


_tc_out = pl.pallas_call(
    _tc_out_body,
    out_shape=jax.ShapeDtypeStruct((N, D), jnp.float32),
)


def kernel(x, edge_index, W1l, b1, W1r, W2l, b2, W2r):
    src = edge_index[0].astype(jnp.int32).reshape(NW, NBLK, IB, CHUNK)
    dst = edge_index[1].astype(jnp.int32).reshape(NW, NBLK, IB, CHUNK)
    zeros_tab = jnp.zeros((NP, D), jnp.float32)
    zeros_cnt = jnp.zeros((NP,), jnp.float32)
    ones = jnp.ones((CHUNK,), jnp.float32)

    cnt = _sc_cnt(dst, zeros_cnt, ones)
    rc = (1.0 / jnp.maximum(cnt[:N] + cnt[NP:NP + N], 1.0)).reshape(N, 1)
    xl1, xr1b = _tc_pre(x, W1l, W1r, b1.reshape(1, D))
    agg1 = _sc_agg(xl1, src, dst, zeros_tab)
    xl2, xr2b = _tc_mid(agg1, rc, xr1b, W2l, W2r, b2.reshape(1, D))
    agg2 = _sc_agg(xl2, src, dst, zeros_tab)
    return _tc_out(agg2, rc, xr2b)


# speedup vs baseline: 1.1797x; 1.0125x over previous
import jax
import jax.numpy as jnp
from jax import lax
from jax.experimental import pallas as pl
from jax.experimental.pallas import tpu as pltpu
from jax.experimental.pallas import tpu_sc as plsc

N = 10000
NP = 10240
D = 128
E = 320000
NC = 2
NS = 16
NW = NC * NS
CHUNK = 125
CPW = E // (NW * CHUNK)
IB = 10
NBLK = CPW // IB
ROWS_PER_TILE = NP // NS


def _sc_agg_body(table, src_idx, dst_idx, zeros_tab,
                 agg_out,
                 agg_sh, src_a, dst_a, src_b, dst_b, rows0, rows1,
                 sem0, sem1, idx_sem):
    c = lax.axis_index("c")
    s = lax.axis_index("s")
    w = c * NS + s

    src_blks = (src_a, src_b)
    dst_blks = (dst_a, dst_b)
    rows = (rows0, rows1)
    sems = (sem0, sem1)

    def idx_refill(m, p, start):
        op = pltpu.async_copy if start else (
            lambda s_, d_, m_: pltpu.make_async_copy(s_, d_, m_).wait())
        op(src_idx.at[w, m], src_blks[p], idx_sem)
        op(dst_idx.at[w, m], dst_blks[p], idx_sem)

    def gather(p, jj, b, start):
        src_ref = table.at[src_blks[p].at[jj]]
        if start:
            pltpu.async_copy(src_ref, rows[b], sems[b])
        else:
            pltpu.make_async_copy(src_ref, rows[b], sems[b]).wait()

    idx_refill(0, 0, True)
    idx_refill(0, 0, False)
    gather(0, 0, 0, True)
    gather(0, 1, 1, True)
    idx_refill(1, 1, True)

    r0 = s * ROWS_PER_TILE
    pltpu.sync_copy(zeros_tab.at[pl.ds(r0, ROWS_PER_TILE)],
                    agg_sh.at[pl.ds(r0, ROWS_PER_TILE)])
    plsc.subcore_barrier()

    def step(kk, carry):
        for p in range(2):
            m = 2 * kk + p
            for jj in range(IB):
                j = m * IB + jj
                b = jj % 2
                gather(p, jj, b, False)
                pltpu.sync_copy(rows[b], agg_sh.at[dst_blks[p].at[jj]],
                                add=True)
                if jj == IB - 2:
                    @pl.when(m + 1 < NBLK)
                    def _():
                        idx_refill(m + 1, 1 - p, False)
                if jj + 2 < IB:
                    @pl.when(j + 2 < CPW)
                    def _():
                        gather(p, jj + 2, b, True)
                else:
                    @pl.when(j + 2 < CPW)
                    def _():
                        gather(1 - p, jj + 2 - IB, b, True)
            @pl.when(m + 2 < NBLK)
            def _():
                idx_refill(m + 2, p, True)
        return carry

    lax.fori_loop(0, NBLK // 2, step, 0)

    plsc.subcore_barrier()

    pltpu.sync_copy(agg_sh.at[pl.ds(r0, ROWS_PER_TILE)],
                    agg_out.at[c, pl.ds(r0, ROWS_PER_TILE)])


_sc_agg = pl.kernel(
    _sc_agg_body,
    out_type=jax.ShapeDtypeStruct((NC, NP, D), jnp.float32),
    mesh=plsc.VectorSubcoreMesh(core_axis_name="c", subcore_axis_name="s"),
    scratch_types=[
        pltpu.VMEM_SHARED((NP, D), jnp.float32),
        pltpu.VMEM((IB, CHUNK), jnp.int32),
        pltpu.VMEM((IB, CHUNK), jnp.int32),
        pltpu.VMEM((IB, CHUNK), jnp.int32),
        pltpu.VMEM((IB, CHUNK), jnp.int32),
        pltpu.VMEM((CHUNK, D), jnp.float32),
        pltpu.VMEM((CHUNK, D), jnp.float32),
        pltpu.SemaphoreType.DMA,
        pltpu.SemaphoreType.DMA,
        pltpu.SemaphoreType.DMA,
    ],
)


def _sc_cnt_body(dst_idx, zeros_cnt, ones_hbm,
                 cnt_out,
                 cnt_sh, dst_v, ones_v, sem):
    c = lax.axis_index("c")
    s = lax.axis_index("s")
    w = c * NS + s

    pltpu.sync_copy(dst_idx.at[w], dst_v)
    pltpu.sync_copy(ones_hbm, ones_v)
    r0 = s * ROWS_PER_TILE
    pltpu.sync_copy(zeros_cnt.at[pl.ds(r0, ROWS_PER_TILE)],
                    cnt_sh.at[pl.ds(r0, ROWS_PER_TILE)])
    plsc.subcore_barrier()

    def step(m, carry):
        for jj in range(IB):
            pltpu.async_copy(ones_v, cnt_sh.at[dst_v.at[m, jj]], sem, add=True)
        for jj in range(IB):
            pltpu.make_async_copy(ones_v, cnt_sh.at[dst_v.at[m, jj]], sem).wait()
        return carry

    lax.fori_loop(0, NBLK, step, 0)

    plsc.subcore_barrier()
    pltpu.sync_copy(cnt_sh.at[pl.ds(r0, ROWS_PER_TILE)],
                    cnt_out.at[pl.ds(c * NP + r0, ROWS_PER_TILE)])


_sc_cnt = pl.kernel(
    _sc_cnt_body,
    out_type=jax.ShapeDtypeStruct((NC * NP,), jnp.float32),
    mesh=plsc.VectorSubcoreMesh(core_axis_name="c", subcore_axis_name="s"),
    scratch_types=[
        pltpu.VMEM_SHARED((NP,), jnp.float32),
        pltpu.VMEM((NBLK, IB, CHUNK), jnp.int32),
        pltpu.VMEM((CHUNK,), jnp.float32),
        pltpu.SemaphoreType.DMA,
    ],
)


_NT = (((1,), (1,)), ((), ()))


def _tc_pre_body(x_ref, wl_ref, wr_ref, b_ref, xl_ref, xrb_ref):
    x = x_ref[...]
    xl_ref[:N] = lax.dot_general(x, wl_ref[...], _NT,
                                 preferred_element_type=jnp.float32)
    xrb_ref[:N] = lax.dot_general(x, wr_ref[...], _NT,
                                  preferred_element_type=jnp.float32) + b_ref[...]


_tc_pre = pl.pallas_call(
    _tc_pre_body,
    out_shape=(jax.ShapeDtypeStruct((NP, D), jnp.float32),
               jax.ShapeDtypeStruct((NP, D), jnp.float32)),
)


def _tc_mid_body(agg_ref, rc_ref, xrb_ref, wl_ref, wr_ref, b_ref,
                 xl_ref, xrb2_ref):
    t = (agg_ref[0, :N] + agg_ref[1, :N]) * rc_ref[...] + xrb_ref[:N]
    h = jnp.maximum(t, 0.0)
    xl_ref[:N] = lax.dot_general(h, wl_ref[...], _NT,
                                 preferred_element_type=jnp.float32)
    xrb2_ref[:N] = lax.dot_general(h, wr_ref[...], _NT,
                                   preferred_element_type=jnp.float32) + b_ref[...]


_tc_mid = pl.pallas_call(
    _tc_mid_body,
    out_shape=(jax.ShapeDtypeStruct((NP, D), jnp.float32),
               jax.ShapeDtypeStruct((NP, D), jnp.float32)),
)


def _tc_out_body(agg_ref, rc_ref, xrb_ref, out_ref):
    out_ref[...] = (agg_ref[0, :N] + agg_ref[1, :N]) * rc_ref[...] + xrb_ref[:N]


_tc_out = pl.pallas_call(
    _tc_out_body,
    out_shape=jax.ShapeDtypeStruct((N, D), jnp.float32),
)


def kernel(x, edge_index, W1l, b1, W1r, W2l, b2, W2r):
    src = edge_index[0].astype(jnp.int32).reshape(NW, NBLK, IB, CHUNK)
    dst = edge_index[1].astype(jnp.int32).reshape(NW, NBLK, IB, CHUNK)
    zeros_tab = jnp.zeros((NP, D), jnp.float32)
    zeros_cnt = jnp.zeros((NP,), jnp.float32)
    ones = jnp.ones((CHUNK,), jnp.float32)

    cnt = _sc_cnt(dst, zeros_cnt, ones)
    rc = (1.0 / jnp.maximum(cnt[:N] + cnt[NP:NP + N], 1.0)).reshape(N, 1)
    xl1, xr1b = _tc_pre(x, W1l, W1r, b1.reshape(1, D))
    agg1 = _sc_agg(xl1, src, dst, zeros_tab)
    xl2, xr2b = _tc_mid(agg1, rc, xr1b, W2l, W2r, b2.reshape(1, D))
    agg2 = _sc_agg(xl2, src, dst, zeros_tab)
    return _tc_out(agg2, rc, xr2b)
